# baseline (speedup 1.0000x reference)
import functools

import jax
import jax.numpy as jnp
from jax import lax
from jax.experimental import pallas as pl
from jax.experimental.pallas import tpu as pltpu
from jax.experimental.pallas import tpu_sc as plsc


def _conv2d(x, w, b, stride, pad):
    out = jax.lax.conv_general_dilated(
        x, w, (stride, stride), [(pad, pad), (pad, pad)],
        dimension_numbers=('NCHW', 'OIHW', 'NCHW'))
    return out + b[None, :, None, None]


def _conv_t2d(x, w, b, stride, pad, out_pad):
    wt = jnp.flip(jnp.transpose(w, (1, 0, 2, 3)), axis=(2, 3))
    k = w.shape[2]
    out = jax.lax.conv_general_dilated(
        x, wt, (1, 1), [(k - 1 - pad, k - 1 - pad + out_pad)] * 2,
        lhs_dilation=(stride, stride), dimension_numbers=('NCHW', 'OIHW', 'NCHW'))
    return out + b[None, :, None, None]


def _group_norm(x, g, bta, groups=2, eps=1e-5):
    B, C, H, W = x.shape
    xg = x.reshape(B, groups, C // groups, H, W)
    m = xg.mean(axis=(2, 3, 4), keepdims=True)
    v = ((xg - m) ** 2).mean(axis=(2, 3, 4), keepdims=True)
    xn = ((xg - m) / jnp.sqrt(v + eps)).reshape(B, C, H, W)
    return xn * g[None, :, None, None] + bta[None, :, None, None]


def _lrelu(x):
    return jnp.where(x >= 0, x, 0.2 * x)


def _topk_body(flat_ref, emb_ref, rown_ref, embn_ref, idx_ref, loss_ref,
               perp_ref, counts_acc, loss_acc, *, n_rows, n_steps,
               commitment_cost):
    step = pl.program_id(0)
    flat = flat_ref[...]
    emb = emb_ref[...]
    R = flat.shape[0]
    K = emb.shape[0]

    dot = jax.lax.dot_general(
        flat, emb, (((1,), (1,)), ((), ())),
        preferred_element_type=jnp.float32)
    rown = rown_ref[...]
    embn = embn_ref[...]
    d = (rown + embn) - 2.0 * dot

    lane = jax.lax.broadcasted_iota(jnp.int32, (R, K), 1)

    def pick(dc):
        m = jnp.min(dc, axis=1, keepdims=True)
        i = jnp.min(jnp.where(dc == m, lane, K), axis=1, keepdims=True)
        return m, i

    m0, i0 = pick(d)
    d = jnp.where(lane == i0, jnp.inf, d)
    _, i1 = pick(d)
    d = jnp.where(lane == i1, jnp.inf, d)
    _, i2 = pick(d)

    idx_ref[...] = i0

    block_loss = jnp.sum(m0)
    block_counts = jnp.sum((lane == i2).astype(jnp.float32), axis=0)[None, :]

    @pl.when(step == 0)
    def _init():
        loss_acc[...] = jnp.zeros_like(loss_acc)
        counts_acc[...] = jnp.zeros_like(counts_acc)

    loss_acc[...] += block_loss.reshape(1, 1)
    counts_acc[...] += block_counts

    @pl.when(step == n_steps - 1)
    def _fin():
        n_elems = jnp.float32(n_rows) * jnp.float32(flat.shape[1])
        loss_ref[...] = commitment_cost * loss_acc[...] / n_elems
        p = counts_acc[...] / jnp.float32(n_rows)
        ent = -jnp.sum(p * jnp.log(p + 1e-10))
        perp_ref[...] = jnp.exp(ent).reshape(1, 1)


def _topk_stage(flat, emb, commitment_cost):
    n_rows, ed = flat.shape
    K = emb.shape[0]
    R = 512
    n_steps = n_rows // R
    assert n_steps * R == n_rows

    body = functools.partial(_topk_body, n_rows=n_rows, n_steps=n_steps,
                             commitment_cost=commitment_cost)
    idx, loss, perp = pl.pallas_call(
        body,
        grid=(n_steps,),
        in_specs=[
            pl.BlockSpec((R, ed), lambda i: (i, 0)),
            pl.BlockSpec((K, ed), lambda i: (0, 0)),
            pl.BlockSpec((R, 1), lambda i: (i, 0)),
            pl.BlockSpec((1, K), lambda i: (0, 0)),
        ],
        out_specs=[
            pl.BlockSpec((R, 1), lambda i: (i, 0)),
            pl.BlockSpec((1, 1), lambda i: (0, 0)),
            pl.BlockSpec((1, 1), lambda i: (0, 0)),
        ],
        out_shape=[
            jax.ShapeDtypeStruct((n_rows, 1), jnp.int32),
            jax.ShapeDtypeStruct((1, 1), jnp.float32),
            jax.ShapeDtypeStruct((1, 1), jnp.float32),
        ],
        scratch_shapes=[
            pltpu.VMEM((1, K), jnp.float32),
            pltpu.VMEM((1, 1), jnp.float32),
        ],
    )(flat, emb,
      jnp.sum(flat ** 2, axis=1, keepdims=True),
      jnp.sum(emb ** 2, axis=1)[None, :])
    return idx[:, 0], loss[0, 0], perp[0, 0]


def _sc_gather(emb, idx):
    NC, NS = 2, 16
    NW = NC * NS
    N = idx.shape[0]
    D = emb.shape[1]
    b_per_w = N // NW
    CH = 112
    NCH = b_per_w // CH
    assert CH * NCH == b_per_w and b_per_w * NW == N and b_per_w % 8 == 0

    mesh = plsc.VectorSubcoreMesh(core_axis_name="c", subcore_axis_name="s")

    @functools.partial(
        pl.kernel, mesh=mesh,
        out_type=jax.ShapeDtypeStruct((N, D), jnp.float32),
        scratch_types=[
            pltpu.VMEM((NCH, CH), jnp.int32),
            pltpu.VMEM((b_per_w, D), jnp.float32),
            pltpu.SemaphoreType.DMA,
        ],
    )
    def k(table_hbm, idx_hbm, out_hbm, idx_v, rows_v, sem):
        wid = lax.axis_index("s") * NC + lax.axis_index("c")
        base = wid * b_per_w
        pltpu.sync_copy(idx_hbm.at[wid], idx_v)
        copies = [
            pltpu.async_copy(table_hbm.at[idx_v.at[j]],
                             rows_v.at[pl.ds(j * CH, CH)], sem)
            for j in range(NCH)
        ]
        for c in copies:
            c.wait()
        pltpu.sync_copy(rows_v, out_hbm.at[pl.ds(base, b_per_w)])

    return k(emb, idx.reshape(NW, NCH, CH))


def kernel(x, enc0_w, enc0_b, enc0_g, enc0_be, enc1_w, enc1_b, enc1_g,
           enc1_be, prevq_w, prevq_b, emb, dec0_w, dec0_b, dec0_g, dec0_be,
           dec1_w, dec1_b, dec1_g, dec1_be, proj_w, proj_b, ro_w, ro_b):
    commitment_cost = 0.25
    enc1 = _lrelu(_group_norm(_conv2d(x, enc0_w, enc0_b, 1, 1), enc0_g, enc0_be))
    latent = _lrelu(_group_norm(_conv2d(enc1, enc1_w, enc1_b, 2, 1), enc1_g, enc1_be))
    z = _conv2d(latent, prevq_w, prevq_b, 1, 0)
    inputs = jnp.transpose(z, (0, 2, 3, 1))
    flat = inputs.reshape(-1, inputs.shape[-1])
    idx0, _kloss, perplexity = _topk_stage(flat, emb, commitment_cost)
    quant_flat = _sc_gather(emb, idx0)
    quant_nhwc = quant_flat.reshape(inputs.shape)
    e_latent_loss = jnp.mean((jax.lax.stop_gradient(quant_nhwc) - inputs) ** 2)
    loss = commitment_cost * e_latent_loss
    quantized = inputs + jax.lax.stop_gradient(quant_nhwc - inputs)
    qz = jnp.transpose(quantized, (0, 3, 1, 2))
    skip_p = _conv2d(enc1, proj_w, proj_b, 1, 0)
    hid = _lrelu(_group_norm(_conv_t2d(qz, dec0_w, dec0_b, 2, 1, 1), dec0_g, dec0_be))
    cat = jnp.concatenate([hid, skip_p], axis=1)
    y = _lrelu(_group_norm(_conv2d(cat, dec1_w, dec1_b, 1, 1), dec1_g, dec1_be))
    x_recon = _conv2d(y, ro_w, ro_b, 1, 0)
    return (loss, x_recon, perplexity)


# speedup vs baseline: 1.2563x; 1.2563x over previous
import functools

import jax
import jax.numpy as jnp
from jax import lax
from jax.experimental import pallas as pl
from jax.experimental.pallas import tpu as pltpu
from jax.experimental.pallas import tpu_sc as plsc


def _conv2d(x, w, b, stride, pad):
    out = jax.lax.conv_general_dilated(
        x, w, (stride, stride), [(pad, pad), (pad, pad)],
        dimension_numbers=('NCHW', 'OIHW', 'NCHW'))
    return out + b[None, :, None, None]


def _conv_t2d(x, w, b, stride, pad, out_pad):
    wt = jnp.flip(jnp.transpose(w, (1, 0, 2, 3)), axis=(2, 3))
    k = w.shape[2]
    out = jax.lax.conv_general_dilated(
        x, wt, (1, 1), [(k - 1 - pad, k - 1 - pad + out_pad)] * 2,
        lhs_dilation=(stride, stride), dimension_numbers=('NCHW', 'OIHW', 'NCHW'))
    return out + b[None, :, None, None]


def _group_norm(x, g, bta, groups=2, eps=1e-5):
    B, C, H, W = x.shape
    xg = x.reshape(B, groups, C // groups, H, W)
    m = xg.mean(axis=(2, 3, 4), keepdims=True)
    v = ((xg - m) ** 2).mean(axis=(2, 3, 4), keepdims=True)
    xn = ((xg - m) / jnp.sqrt(v + eps)).reshape(B, C, H, W)
    return xn * g[None, :, None, None] + bta[None, :, None, None]


def _lrelu(x):
    return jnp.where(x >= 0, x, 0.2 * x)


def _topk_body(flat_ref, emb_ref, rown_ref, embn_ref, idx_ref, loss_ref,
               perp_ref, counts_acc, loss_acc, *, n_rows, n_steps,
               commitment_cost):
    step = pl.program_id(0)
    flat = flat_ref[...]
    emb = emb_ref[...]
    R = flat.shape[0]
    K = emb.shape[0]

    dot = jax.lax.dot_general(
        flat, emb, (((1,), (1,)), ((), ())),
        preferred_element_type=jnp.float32)
    rown = rown_ref[...]
    embn = embn_ref[...]
    d = (rown + embn) - 2.0 * dot

    lane = jax.lax.broadcasted_iota(jnp.int32, (R, K), 1)

    def pick(dc):
        m = jnp.min(dc, axis=1, keepdims=True)
        i = jnp.min(jnp.where(dc == m, lane, K), axis=1, keepdims=True)
        return m, i

    m0, i0 = pick(d)
    d = jnp.where(lane == i0, jnp.inf, d)
    _, i1 = pick(d)
    d = jnp.where(lane == i1, jnp.inf, d)
    _, i2 = pick(d)

    idx_ref[...] = i0

    block_loss = jnp.sum(m0)
    block_counts = jnp.sum((lane == i2).astype(jnp.float32), axis=0)[None, :]

    @pl.when(step == 0)
    def _init():
        loss_acc[...] = jnp.zeros_like(loss_acc)
        counts_acc[...] = jnp.zeros_like(counts_acc)

    loss_acc[...] += block_loss.reshape(1, 1)
    counts_acc[...] += block_counts

    @pl.when(step == n_steps - 1)
    def _fin():
        n_elems = jnp.float32(n_rows) * jnp.float32(flat.shape[1])
        loss_ref[...] = commitment_cost * loss_acc[...] / n_elems
        p = counts_acc[...] / jnp.float32(n_rows)
        ent = -jnp.sum(p * jnp.log(p + 1e-10))
        perp_ref[...] = jnp.exp(ent).reshape(1, 1)


def _topk_stage(flat, emb, commitment_cost):
    n_rows, ed = flat.shape
    K = emb.shape[0]
    R = 512
    n_steps = n_rows // R
    assert n_steps * R == n_rows

    body = functools.partial(_topk_body, n_rows=n_rows, n_steps=n_steps,
                             commitment_cost=commitment_cost)
    idx, loss, perp = pl.pallas_call(
        body,
        grid=(n_steps,),
        in_specs=[
            pl.BlockSpec((R, ed), lambda i: (i, 0)),
            pl.BlockSpec((K, ed), lambda i: (0, 0)),
            pl.BlockSpec((R, 1), lambda i: (i, 0)),
            pl.BlockSpec((1, K), lambda i: (0, 0)),
        ],
        out_specs=[
            pl.BlockSpec((R, 1), lambda i: (i, 0)),
            pl.BlockSpec((1, 1), lambda i: (0, 0)),
            pl.BlockSpec((1, 1), lambda i: (0, 0)),
        ],
        out_shape=[
            jax.ShapeDtypeStruct((n_rows, 1), jnp.int32),
            jax.ShapeDtypeStruct((1, 1), jnp.float32),
            jax.ShapeDtypeStruct((1, 1), jnp.float32),
        ],
        scratch_shapes=[
            pltpu.VMEM((1, K), jnp.float32),
            pltpu.VMEM((1, 1), jnp.float32),
        ],
    )(flat, emb,
      jnp.sum(flat ** 2, axis=1, keepdims=True),
      jnp.sum(emb ** 2, axis=1)[None, :])
    return idx[:, 0], loss[0, 0], perp[0, 0]


def _sc_gather(emb, idx):
    NC, NS = 2, 16
    NW = NC * NS
    N = idx.shape[0]
    D = emb.shape[1]
    b_per_w = N // NW
    CH = 112
    NCH = b_per_w // CH
    K_ROWS = emb.shape[0]
    assert CH * NCH == b_per_w and b_per_w * NW == N and b_per_w % 8 == 0

    mesh = plsc.VectorSubcoreMesh(core_axis_name="c", subcore_axis_name="s")

    @functools.partial(
        pl.kernel, mesh=mesh,
        out_type=jax.ShapeDtypeStruct((N, D), jnp.float32),
        scratch_types=[
            pltpu.VMEM((NCH, CH), jnp.int32),
            pltpu.VMEM((b_per_w, D), jnp.float32),
            pltpu.VMEM_SHARED((K_ROWS, D), jnp.float32),
            pltpu.SemaphoreType.DMA,
        ],
    )
    def k(table_hbm, idx_hbm, out_hbm, idx_v, rows_v, emb_sh, sem):
        wid = lax.axis_index("s") * NC + lax.axis_index("c")
        base = wid * b_per_w
        @pl.when(lax.axis_index("s") == 0)
        def _stage():
            pltpu.sync_copy(table_hbm, emb_sh)
        plsc.subcore_barrier()
        pltpu.sync_copy(idx_hbm.at[wid], idx_v)
        copies = [
            pltpu.async_copy(emb_sh.at[idx_v.at[j]],
                             rows_v.at[pl.ds(j * CH, CH)], sem)
            for j in range(NCH)
        ]
        for c in copies:
            c.wait()
        pltpu.sync_copy(rows_v, out_hbm.at[pl.ds(base, b_per_w)])

    return k(emb, idx.reshape(NW, NCH, CH))


def kernel(x, enc0_w, enc0_b, enc0_g, enc0_be, enc1_w, enc1_b, enc1_g,
           enc1_be, prevq_w, prevq_b, emb, dec0_w, dec0_b, dec0_g, dec0_be,
           dec1_w, dec1_b, dec1_g, dec1_be, proj_w, proj_b, ro_w, ro_b):
    commitment_cost = 0.25
    enc1 = _lrelu(_group_norm(_conv2d(x, enc0_w, enc0_b, 1, 1), enc0_g, enc0_be))
    latent = _lrelu(_group_norm(_conv2d(enc1, enc1_w, enc1_b, 2, 1), enc1_g, enc1_be))
    z = _conv2d(latent, prevq_w, prevq_b, 1, 0)
    inputs = jnp.transpose(z, (0, 2, 3, 1))
    flat = inputs.reshape(-1, inputs.shape[-1])
    idx0, _kloss, perplexity = _topk_stage(flat, emb, commitment_cost)
    quant_flat = _sc_gather(emb, idx0)
    quant_nhwc = quant_flat.reshape(inputs.shape)
    e_latent_loss = jnp.mean((jax.lax.stop_gradient(quant_nhwc) - inputs) ** 2)
    loss = commitment_cost * e_latent_loss
    quantized = inputs + jax.lax.stop_gradient(quant_nhwc - inputs)
    qz = jnp.transpose(quantized, (0, 3, 1, 2))
    skip_p = _conv2d(enc1, proj_w, proj_b, 1, 0)
    hid = _lrelu(_group_norm(_conv_t2d(qz, dec0_w, dec0_b, 2, 1, 1), dec0_g, dec0_be))
    cat = jnp.concatenate([hid, skip_p], axis=1)
    y = _lrelu(_group_norm(_conv2d(cat, dec1_w, dec1_b, 1, 1), dec1_g, dec1_be))
    x_recon = _conv2d(y, ro_w, ro_b, 1, 0)
    return (loss, x_recon, perplexity)


# speedup vs baseline: 1.2715x; 1.0122x over previous
import functools

import jax
import jax.numpy as jnp
from jax import lax
from jax.experimental import pallas as pl
from jax.experimental.pallas import tpu as pltpu
from jax.experimental.pallas import tpu_sc as plsc


def _conv2d(x, w, b, stride, pad):
    out = jax.lax.conv_general_dilated(
        x, w, (stride, stride), [(pad, pad), (pad, pad)],
        dimension_numbers=('NCHW', 'OIHW', 'NCHW'))
    return out + b[None, :, None, None]


def _conv_t2d(x, w, b, stride, pad, out_pad):
    wt = jnp.flip(jnp.transpose(w, (1, 0, 2, 3)), axis=(2, 3))
    k = w.shape[2]
    out = jax.lax.conv_general_dilated(
        x, wt, (1, 1), [(k - 1 - pad, k - 1 - pad + out_pad)] * 2,
        lhs_dilation=(stride, stride), dimension_numbers=('NCHW', 'OIHW', 'NCHW'))
    return out + b[None, :, None, None]


def _group_norm(x, g, bta, groups=2, eps=1e-5):
    B, C, H, W = x.shape
    xg = x.reshape(B, groups, C // groups, H, W)
    m = xg.mean(axis=(2, 3, 4), keepdims=True)
    v = ((xg - m) ** 2).mean(axis=(2, 3, 4), keepdims=True)
    xn = ((xg - m) / jnp.sqrt(v + eps)).reshape(B, C, H, W)
    return xn * g[None, :, None, None] + bta[None, :, None, None]


def _lrelu(x):
    return jnp.where(x >= 0, x, 0.2 * x)


def _topk_body(flat_ref, emb_ref, rown_ref, embn_ref, idx_ref, loss_ref,
               perp_ref, counts_acc, loss_acc, *, n_rows, n_steps,
               commitment_cost):
    step = pl.program_id(0)
    flat = flat_ref[...]
    emb = emb_ref[...]
    R = flat.shape[0]
    K = emb.shape[0]

    dot = jax.lax.dot_general(
        flat, emb, (((1,), (1,)), ((), ())),
        preferred_element_type=jnp.float32)
    rown = rown_ref[...]
    embn = embn_ref[...]
    d = (rown + embn) - 2.0 * dot

    lanef = jax.lax.broadcasted_iota(jnp.int32, (R, K), 1).astype(jnp.float32)
    bigf = jnp.float32(K)

    def pick(dc):
        m = jnp.min(dc, axis=1, keepdims=True)
        i = jnp.min(jnp.where(dc == m, lanef, bigf), axis=1, keepdims=True)
        return m, i

    m0, i0 = pick(d)
    d = jnp.where(lanef == i0, jnp.inf, d)
    _, i1 = pick(d)
    d = jnp.where(lanef == i1, jnp.inf, d)
    _, i2 = pick(d)

    idx_ref[...] = i0.astype(jnp.int32)

    block_loss = jnp.sum(m0)
    block_counts = jnp.sum((lanef == i2).astype(jnp.float32), axis=0)[None, :]

    @pl.when(step == 0)
    def _init():
        loss_acc[...] = jnp.zeros_like(loss_acc)
        counts_acc[...] = jnp.zeros_like(counts_acc)

    loss_acc[...] += block_loss.reshape(1, 1)
    counts_acc[...] += block_counts

    @pl.when(step == n_steps - 1)
    def _fin():
        n_elems = jnp.float32(n_rows) * jnp.float32(flat.shape[1])
        loss_ref[...] = commitment_cost * loss_acc[...] / n_elems
        p = counts_acc[...] / jnp.float32(n_rows)
        ent = -jnp.sum(p * jnp.log(p + 1e-10))
        perp_ref[...] = jnp.exp(ent).reshape(1, 1)


def _topk_stage(flat, emb, commitment_cost):
    n_rows, ed = flat.shape
    K = emb.shape[0]
    R = 512
    n_steps = n_rows // R
    assert n_steps * R == n_rows

    body = functools.partial(_topk_body, n_rows=n_rows, n_steps=n_steps,
                             commitment_cost=commitment_cost)
    idx, loss, perp = pl.pallas_call(
        body,
        grid=(n_steps,),
        in_specs=[
            pl.BlockSpec((R, ed), lambda i: (i, 0)),
            pl.BlockSpec((K, ed), lambda i: (0, 0)),
            pl.BlockSpec((R, 1), lambda i: (i, 0)),
            pl.BlockSpec((1, K), lambda i: (0, 0)),
        ],
        out_specs=[
            pl.BlockSpec((R, 1), lambda i: (i, 0)),
            pl.BlockSpec((1, 1), lambda i: (0, 0)),
            pl.BlockSpec((1, 1), lambda i: (0, 0)),
        ],
        out_shape=[
            jax.ShapeDtypeStruct((n_rows, 1), jnp.int32),
            jax.ShapeDtypeStruct((1, 1), jnp.float32),
            jax.ShapeDtypeStruct((1, 1), jnp.float32),
        ],
        scratch_shapes=[
            pltpu.VMEM((1, K), jnp.float32),
            pltpu.VMEM((1, 1), jnp.float32),
        ],
    )(flat, emb,
      jnp.sum(flat ** 2, axis=1, keepdims=True),
      jnp.sum(emb ** 2, axis=1)[None, :])
    return idx[:, 0], loss[0, 0], perp[0, 0]


def _sc_gather(emb, idx):
    NC, NS = 2, 16
    NW = NC * NS
    N = idx.shape[0]
    D = emb.shape[1]
    b_per_w = N // NW
    CH = 112
    NCH = b_per_w // CH
    K_ROWS = emb.shape[0]
    assert CH * NCH == b_per_w and b_per_w * NW == N and b_per_w % 8 == 0

    mesh = plsc.VectorSubcoreMesh(core_axis_name="c", subcore_axis_name="s")

    @functools.partial(
        pl.kernel, mesh=mesh,
        out_type=jax.ShapeDtypeStruct((N, D), jnp.float32),
        scratch_types=[
            pltpu.VMEM((NCH, CH), jnp.int32),
            pltpu.VMEM((b_per_w, D), jnp.float32),
            pltpu.VMEM_SHARED((K_ROWS, D), jnp.float32),
            pltpu.SemaphoreType.DMA,
        ],
    )
    def k(table_hbm, idx_hbm, out_hbm, idx_v, rows_v, emb_sh, sem):
        wid = lax.axis_index("s") * NC + lax.axis_index("c")
        base = wid * b_per_w
        @pl.when(lax.axis_index("s") == 0)
        def _stage():
            pltpu.sync_copy(table_hbm, emb_sh)
        plsc.subcore_barrier()
        pltpu.sync_copy(idx_hbm.at[wid], idx_v)
        copies = [
            pltpu.async_copy(emb_sh.at[idx_v.at[j]],
                             rows_v.at[pl.ds(j * CH, CH)], sem)
            for j in range(NCH)
        ]
        for c in copies:
            c.wait()
        pltpu.sync_copy(rows_v, out_hbm.at[pl.ds(base, b_per_w)])

    return k(emb, idx.reshape(NW, NCH, CH))


def kernel(x, enc0_w, enc0_b, enc0_g, enc0_be, enc1_w, enc1_b, enc1_g,
           enc1_be, prevq_w, prevq_b, emb, dec0_w, dec0_b, dec0_g, dec0_be,
           dec1_w, dec1_b, dec1_g, dec1_be, proj_w, proj_b, ro_w, ro_b):
    commitment_cost = 0.25
    enc1 = _lrelu(_group_norm(_conv2d(x, enc0_w, enc0_b, 1, 1), enc0_g, enc0_be))
    latent = _lrelu(_group_norm(_conv2d(enc1, enc1_w, enc1_b, 2, 1), enc1_g, enc1_be))
    z = _conv2d(latent, prevq_w, prevq_b, 1, 0)
    inputs = jnp.transpose(z, (0, 2, 3, 1))
    flat = inputs.reshape(-1, inputs.shape[-1])
    idx0, _kloss, perplexity = _topk_stage(flat, emb, commitment_cost)
    quant_flat = _sc_gather(emb, idx0)
    quant_nhwc = quant_flat.reshape(inputs.shape)
    e_latent_loss = jnp.mean((jax.lax.stop_gradient(quant_nhwc) - inputs) ** 2)
    loss = commitment_cost * e_latent_loss
    quantized = inputs + jax.lax.stop_gradient(quant_nhwc - inputs)
    qz = jnp.transpose(quantized, (0, 3, 1, 2))
    skip_p = _conv2d(enc1, proj_w, proj_b, 1, 0)
    hid = _lrelu(_group_norm(_conv_t2d(qz, dec0_w, dec0_b, 2, 1, 1), dec0_g, dec0_be))
    cat = jnp.concatenate([hid, skip_p], axis=1)
    y = _lrelu(_group_norm(_conv2d(cat, dec1_w, dec1_b, 1, 1), dec1_g, dec1_be))
    x_recon = _conv2d(y, ro_w, ro_b, 1, 0)
    return (loss, x_recon, perplexity)
